# baseline (speedup 1.0000x reference)
import jax
import jax.numpy as jnp
from jax.experimental import pallas as pl
from jax.experimental.pallas import tpu as pltpu

N = 10000
E = 160000
DIM_H = 256
H = 8
DH = 32


def _finalize_body(x_ref, wv_ref, z_ref, gamma_ref, beta_ref, out_ref):
    x = x_ref[...]
    wv = wv_ref[...]
    z = z_ref[...]
    h = x + wv / (z + 1e-6)
    mean = jnp.mean(h, axis=0, keepdims=True)
    var = jnp.mean((h - mean) ** 2, axis=0, keepdims=True)
    out_ref[...] = (h - mean) / jnp.sqrt(var + 1e-5) * gamma_ref[...] + beta_ref[...]


def kernel(x, edge_index, edge_attr, WQ, WK, WV, WE, gamma, beta):
    n, dim_h = x.shape
    Q = (x @ WQ).reshape(n, H, DH)
    K = (x @ WK).reshape(n, H, DH)
    V = (x @ WV).reshape(n, H, DH)
    Ef = (edge_attr @ WE).reshape(-1, H, DH)
    src = edge_index[0]
    dst = edge_index[1]
    score = jnp.take(K, src, axis=0) * jnp.take(Q, dst, axis=0)
    score = score / jnp.sqrt(jnp.float32(DH))
    score = score * Ef
    score = jnp.exp(jnp.clip(jnp.sum(score, axis=-1, keepdims=True), -5.0, 5.0))
    msg = jnp.take(V, src, axis=0) * score
    wV = jax.ops.segment_sum(msg, dst, num_segments=n)
    Z = jax.ops.segment_sum(score, dst, num_segments=n)
    zfull = jnp.repeat(Z.reshape(n, H), DH, axis=1)
    out = pl.pallas_call(
        _finalize_body,
        out_shape=jax.ShapeDtypeStruct((n, dim_h), jnp.float32),
    )(x, wV.reshape(n, dim_h), zfull, gamma.reshape(1, dim_h),
      beta.reshape(1, dim_h))
    return out


# speedup vs baseline: 10.6927x; 10.6927x over previous
import functools

import jax
import jax.numpy as jnp
from jax import lax
from jax.experimental import pallas as pl
from jax.experimental.pallas import tpu as pltpu
from jax.experimental.pallas import tpu_sc as plsc

N = 10000
E = 160000
DIM_H = 256
NH = 8
DH = 32

CHUNK = 32
EDGES_PER_TILE = E // 16
NCHUNK = EDGES_PER_TILE // CHUNK
TAIL = EDGES_PER_TILE - NCHUNK * CHUNK
ROWS_PER_TILE = 624


def _proj_body(x_ref, w_ref, o_ref):
    o_ref[0] = jnp.dot(x_ref[...], w_ref[0], preferred_element_type=jnp.float32)


def _qkv_table(x, WQ, WK, WV):
    wcat = jnp.stack([WK[:, :128], WK[:, 128:], WQ[:, :128], WQ[:, 128:],
                      WV[:, :128], WV[:, 128:]])
    out = pl.pallas_call(
        _proj_body,
        grid=(6, N // 400),
        in_specs=[
            pl.BlockSpec((400, DIM_H), lambda t, rb: (rb, 0)),
            pl.BlockSpec((1, DIM_H, 128), lambda t, rb: (t, 0, 0)),
        ],
        out_specs=pl.BlockSpec((1, 400, 128), lambda t, rb: (t, rb, 0)),
        out_shape=jax.ShapeDtypeStruct((6, N, 128), jnp.float32),
    )(x, wcat)
    return out.reshape(6 * N, 128)


def _ef_table(edge_attr, WE):
    we2 = jnp.stack([WE[:, :128], WE[:, 128:]])
    out = pl.pallas_call(
        _proj_body,
        grid=(2, E // 640),
        in_specs=[
            pl.BlockSpec((640, DIM_H), lambda h, eb: (eb, 0)),
            pl.BlockSpec((1, DIM_H, 128), lambda h, eb: (h, 0, 0)),
        ],
        out_specs=pl.BlockSpec((1, 640, 128), lambda h, eb: (h, eb, 0)),
        out_shape=jax.ShapeDtypeStruct((2, E, 128), jnp.float32),
    )(edge_attr, we2)
    return out.reshape(2 * E, 128)


_ROW_STARTS = [CHUNK * j for j in range(ROWS_PER_TILE // CHUNK)]
if ROWS_PER_TILE % CHUNK:
    _ROW_STARTS.append(ROWS_PER_TILE - CHUNK)


def _sc_edge_body(table_hbm, ef_hbm, src_hbm, dst_hbm, wv_hbm, sco_hbm,
                  acc, kbuf, qbuf, vbuf, efbuf, sbuf,
                  kidx, qidx, vidx, didx, sem):
    hs = lax.axis_index("c")
    tid = lax.axis_index("s")

    zero16f = jnp.zeros((16,), jnp.float32)
    lane = lax.iota(jnp.int32, 16)
    scale = jnp.float32(1.0 / (DH ** 0.5))
    perms = [lane ^ k for k in (8, 4, 2, 1)]
    gdn = lax.GatherDimensionNumbers(
        offset_dims=(), collapsed_slice_dims=(0,), start_index_map=(0,))

    def _hsum(v):
        for p in perms:
            v = v + lax.gather(v, p[:, None], dimension_numbers=gdn,
                               slice_sizes=(1,),
                               mode=lax.GatherScatterMode.PROMISE_IN_BOUNDS)
        return v

    def _zero_kbuf(r, _):
        for j in range(8):
            kbuf[r, pl.ds(16 * j, 16)] = zero16f
        return 0
    lax.fori_loop(0, CHUNK, _zero_kbuf, 0)

    row0 = tid * ROWS_PER_TILE

    def _set_iota(b):
        for j in range(CHUNK // 16):
            qidx[pl.ds(16 * j, 16)] = b + 16 * j + lane

    for start in _ROW_STARTS:
        _set_iota(row0 + start)
        pltpu.sync_copy(kbuf, acc.at[qidx])
    _set_iota(N - CHUNK)
    pltpu.sync_copy(kbuf, acc.at[qidx])
    plsc.subcore_barrier()

    def _process(c, base):
        def _sl(ref):
            return ref if ref.shape[0] == c else ref.at[pl.ds(0, c)]

        pltpu.sync_copy(src_hbm.at[pl.ds(base, c)], _sl(kidx))
        pltpu.sync_copy(dst_hbm.at[pl.ds(base, c)], _sl(didx))
        for j in range(c // 16):
            s = kidx[pl.ds(16 * j, 16)]
            d = didx[pl.ds(16 * j, 16)]
            vidx[pl.ds(16 * j, 16)] = s + (4 * N) + hs * N
            kidx[pl.ds(16 * j, 16)] = s + hs * N
            qidx[pl.ds(16 * j, 16)] = d + (2 * N) + hs * N
        ck = pltpu.async_copy(table_hbm.at[_sl(kidx)], _sl(kbuf), sem)
        cq = pltpu.async_copy(table_hbm.at[_sl(qidx)], _sl(qbuf), sem)
        cv = pltpu.async_copy(table_hbm.at[_sl(vidx)], _sl(vbuf), sem)
        ce = pltpu.async_copy(ef_hbm.at[pl.ds(hs * E + base, c)], _sl(efbuf), sem)
        ck.wait()
        cq.wait()
        cv.wait()
        ce.wait()

        def _edge(e, _):
            svs = []
            for h in range(4):
                p0 = (kbuf[e, pl.ds(32 * h, 16)]
                      * qbuf[e, pl.ds(32 * h, 16)]
                      * efbuf[e, pl.ds(32 * h, 16)])
                p1 = (kbuf[e, pl.ds(32 * h + 16, 16)]
                      * qbuf[e, pl.ds(32 * h + 16, 16)]
                      * efbuf[e, pl.ds(32 * h + 16, 16)])
                s = jnp.clip(_hsum(p0 + p1) * scale, -5.0, 5.0)
                sv = jnp.exp(s)
                svs.append(sv)
                for j2 in range(2):
                    sl = pl.ds(32 * h + 16 * j2, 16)
                    vbuf[e, sl] = vbuf[e, sl] * sv
            row = jnp.zeros((16,), jnp.float32)
            for h in range(4):
                row = jnp.where(lane == h, svs[h], row)
            sbuf[e, :] = row
            return 0
        lax.fori_loop(0, c, _edge, 0)

        pltpu.sync_copy(_sl(sbuf), sco_hbm.at[pl.ds(hs * E + base, c)])

        if c < CHUNK:
            def _zrow(r, _):
                for j in range(8):
                    vbuf[r, pl.ds(16 * j, 16)] = zero16f
                return 0
            lax.fori_loop(c, CHUNK, _zrow, 0)

        pltpu.sync_copy(vbuf, acc.at[didx], add=True)

    def _chunk(i, _):
        _process(CHUNK, tid * EDGES_PER_TILE + i * CHUNK)
        return 0

    lax.fori_loop(0, NCHUNK, _chunk, 0)
    _process(TAIL, tid * EDGES_PER_TILE + NCHUNK * CHUNK)
    plsc.subcore_barrier()

    def _dump(r0):
        _set_iota(r0)
        pltpu.sync_copy(acc.at[qidx], kbuf)
        pltpu.sync_copy(kbuf, wv_hbm.at[pl.ds(hs * N + r0, CHUNK)])

    for start in _ROW_STARTS:
        _dump(row0 + start)
    _dump(N - CHUNK)


def _sc_z_body(dst_hbm, sco_hbm, z_hbm,
               zacc, zbuf, sbuf, didx, qidx, sem):
    hs = lax.axis_index("c")
    tid = lax.axis_index("s")

    zero16f = jnp.zeros((16,), jnp.float32)
    lane = lax.iota(jnp.int32, 16)

    def _zero_zbuf(r, _):
        for j in range(8):
            zbuf[r, pl.ds(16 * j, 16)] = zero16f
        return 0
    lax.fori_loop(0, CHUNK, _zero_zbuf, 0)

    row0 = tid * ROWS_PER_TILE

    def _set_iota(b):
        for j in range(CHUNK // 16):
            qidx[pl.ds(16 * j, 16)] = b + 16 * j + lane

    for start in _ROW_STARTS:
        _set_iota(row0 + start)
        pltpu.sync_copy(zbuf, zacc.at[qidx])
    _set_iota(N - CHUNK)
    pltpu.sync_copy(zbuf, zacc.at[qidx])
    plsc.subcore_barrier()

    def _process(c, base):
        def _sl(ref):
            return ref if ref.shape[0] == c else ref.at[pl.ds(0, c)]

        pltpu.sync_copy(dst_hbm.at[pl.ds(base, c)], _sl(didx))
        pltpu.sync_copy(sco_hbm.at[pl.ds(hs * E + base, c)], _sl(sbuf))

        def _edge(e, _):
            zbuf[e, pl.ds(0, 16)] = sbuf[e, :]
            return 0
        lax.fori_loop(0, c, _edge, 0)

        if c < CHUNK:
            def _zrow(r, _):
                zbuf[r, pl.ds(0, 16)] = zero16f
                return 0
            lax.fori_loop(c, CHUNK, _zrow, 0)

        pltpu.sync_copy(zbuf, zacc.at[didx], add=True)

    def _chunk(i, _):
        _process(CHUNK, tid * EDGES_PER_TILE + i * CHUNK)
        return 0

    lax.fori_loop(0, NCHUNK, _chunk, 0)
    _process(TAIL, tid * EDGES_PER_TILE + NCHUNK * CHUNK)
    plsc.subcore_barrier()

    def _dump(r0):
        _set_iota(r0)
        pltpu.sync_copy(zacc.at[qidx], zbuf)
        pltpu.sync_copy(zbuf, z_hbm.at[pl.ds(hs * N + r0, CHUNK)])

    for start in _ROW_STARTS:
        _dump(row0 + start)
    _dump(N - CHUNK)


def _sc_edge(table, ef, src, dst):
    mesh = plsc.VectorSubcoreMesh(core_axis_name="c", subcore_axis_name="s")
    wv, sco = functools.partial(
        pl.kernel,
        out_type=[jax.ShapeDtypeStruct((2 * N, 128), jnp.float32),
                  jax.ShapeDtypeStruct((2 * E, 16), jnp.float32)],
        mesh=mesh,
        scratch_types=[
            pltpu.VMEM_SHARED((N, 128), jnp.float32),
            pltpu.VMEM((CHUNK, 128), jnp.float32),
            pltpu.VMEM((CHUNK, 128), jnp.float32),
            pltpu.VMEM((CHUNK, 128), jnp.float32),
            pltpu.VMEM((CHUNK, 128), jnp.float32),
            pltpu.VMEM((CHUNK, 16), jnp.float32),
            pltpu.VMEM((CHUNK,), jnp.int32),
            pltpu.VMEM((CHUNK,), jnp.int32),
            pltpu.VMEM((CHUNK,), jnp.int32),
            pltpu.VMEM((CHUNK,), jnp.int32),
            pltpu.SemaphoreType.DMA,
        ],
    )(_sc_edge_body)(table, ef, src, dst)
    z = functools.partial(
        pl.kernel,
        out_type=jax.ShapeDtypeStruct((2 * N, 128), jnp.float32),
        mesh=plsc.VectorSubcoreMesh(core_axis_name="c", subcore_axis_name="s"),
        scratch_types=[
            pltpu.VMEM_SHARED((N, 128), jnp.float32),
            pltpu.VMEM((CHUNK, 128), jnp.float32),
            pltpu.VMEM((CHUNK, 16), jnp.float32),
            pltpu.VMEM((CHUNK,), jnp.int32),
            pltpu.VMEM((CHUNK,), jnp.int32),
            pltpu.SemaphoreType.DMA,
        ],
    )(_sc_z_body)(dst, sco)
    return wv, z


def _final_body(x_ref, wv_ref, z_ref, g_ref, b_ref, o_ref):
    x = x_ref[...]
    wv = wv_ref[...]
    z = z_ref[...]
    parts = []
    for h in range(4):
        zh = z[:, h:h + 1] + 1e-6
        parts.append(wv[:, 32 * h:32 * h + 32] / zh)
    hmat = x + jnp.concatenate(parts, axis=1)
    mean = jnp.mean(hmat, axis=0, keepdims=True)
    var = jnp.mean((hmat - mean) ** 2, axis=0, keepdims=True)
    o_ref[...] = ((hmat - mean) / jnp.sqrt(var + 1e-5)) * g_ref[...] + b_ref[...]


def _finalize(x, wv, z, gamma, beta):
    return pl.pallas_call(
        _final_body,
        grid=(2,),
        in_specs=[
            pl.BlockSpec((N, 128), lambda h: (0, h)),
            pl.BlockSpec((N, 128), lambda h: (h, 0)),
            pl.BlockSpec((N, 128), lambda h: (h, 0)),
            pl.BlockSpec((1, 128), lambda h: (0, h)),
            pl.BlockSpec((1, 128), lambda h: (0, h)),
        ],
        out_specs=pl.BlockSpec((N, 128), lambda h: (0, h)),
        out_shape=jax.ShapeDtypeStruct((N, DIM_H), jnp.float32),
    )(x, wv, z, gamma.reshape(1, DIM_H), beta.reshape(1, DIM_H))


def kernel(x, edge_index, edge_attr, WQ, WK, WV, WE, gamma, beta):
    src = edge_index[0].astype(jnp.int32)
    dst = edge_index[1].astype(jnp.int32)
    table = _qkv_table(x, WQ, WK, WV)
    ef = _ef_table(edge_attr, WE)
    wv, z = _sc_edge(table, ef, src, dst)
    return _finalize(x, wv, z, gamma, beta)
